# Initial kernel scaffold; baseline (speedup 1.0000x reference)
import jax
import jax.numpy as jnp
from jax.experimental import pallas as pl


def kernel(edge_index, edge_weight, source_nodes):
    raise NotImplementedError("write your pallas kernel here")


# speedup vs baseline: 78.3361x; 78.3361x over previous
import dataclasses
import functools

import jax
import jax.numpy as jnp
from jax import lax
from jax.experimental import pallas as pl
from jax.experimental.pallas import tpu as pltpu
from jax.experimental.pallas import tpu_sc as plsc

_N = 100000
_E = 1600000
_ALPHA = 0.85
_ITERS = 100
_NSRC = 256

_LANES = 16
_TILES = 16
_BLK = 2048
_NBLK = 49
_EPT = _BLK * _NBLK
_EPAD = _EPT * _TILES
_NPAD = 100096
_NSLICE = _NPAD // _TILES
_BINIT = (1.0 - _ALPHA) / (_ALPHA * _NSRC)


def _f32x16(v):
    return jnp.full((_LANES,), v, dtype=jnp.float32)


def _body(src_hbm, dst_hbm, ew_hbm, srcn_hbm, xout_hbm, w_hbm,
          x_vmem, srcv, dstv, wv, pv, srcnv, biasv, binitv, slicev, xmsh):
    sid = lax.axis_index("s")
    ebase = sid * _EPT
    nbase = sid * _NSLICE

    @pl.loop(0, _NSLICE // _LANES)
    def _(j):
        slicev[pl.ds(j * _LANES, _LANES)] = _f32x16(0.0)

    pltpu.sync_copy(slicev, xmsh.at[pl.ds(nbase, _NSLICE)])
    plsc.subcore_barrier()

    @pl.loop(0, _NBLK)
    def _(blk):
        e0 = ebase + blk * _BLK
        pltpu.sync_copy(src_hbm.at[pl.ds(e0, _BLK)], srcv)
        pltpu.sync_copy(ew_hbm.at[pl.ds(e0, _BLK)], wv)
        pltpu.sync_copy(wv, xmsh.at[srcv], add=True)
    plsc.subcore_barrier()

    pltpu.sync_copy(xmsh, x_vmem)
    plsc.subcore_barrier()

    pltpu.sync_copy(slicev, xmsh.at[pl.ds(nbase, _NSLICE)])
    plsc.subcore_barrier()

    @pl.when(sid == 0)
    def _():
        pltpu.sync_copy(srcn_hbm, srcnv)

        @pl.loop(0, _NSRC // _LANES)
        def _(k):
            biasv[pl.ds(k * _LANES, _LANES)] = _f32x16(_BINIT)

        pltpu.sync_copy(biasv, xmsh.at[srcnv], add=True)
    plsc.subcore_barrier()

    pltpu.sync_copy(xmsh.at[pl.ds(nbase, _NSLICE)], binitv)

    @pl.loop(0, _NBLK)
    def _(blk):
        e0 = ebase + blk * _BLK
        pltpu.sync_copy(src_hbm.at[pl.ds(e0, _BLK)], srcv)
        pltpu.sync_copy(ew_hbm.at[pl.ds(e0, _BLK)], wv)
        for i in range(_BLK // _LANES):
            sl = pl.ds(i * _LANES, _LANES)
            rs = plsc.load_gather(x_vmem, [srcv[sl]])
            nz = rs != _f32x16(0.0)
            safe = jnp.where(nz, rs, _f32x16(1.0))
            pv[sl] = jnp.where(nz, wv[sl] / safe, _f32x16(0.0))
        pltpu.sync_copy(pv, w_hbm.at[pl.ds(e0, _BLK)])

    @pl.loop(0, _NPAD // _LANES)
    def _(j):
        x_vmem[pl.ds(j * _LANES, _LANES)] = _f32x16(1.0 / _N)
    plsc.subcore_barrier()

    @pl.loop(0, _ITERS)
    def _(it):
        @pl.loop(0, _NBLK)
        def _(blk):
            e0 = ebase + blk * _BLK
            pltpu.sync_copy(src_hbm.at[pl.ds(e0, _BLK)], srcv)
            pltpu.sync_copy(dst_hbm.at[pl.ds(e0, _BLK)], dstv)
            pltpu.sync_copy(w_hbm.at[pl.ds(e0, _BLK)], wv)
            for i in range(_BLK // _LANES):
                sl = pl.ds(i * _LANES, _LANES)
                xg = plsc.load_gather(x_vmem, [srcv[sl]])
                pv[sl] = xg * wv[sl]
            pltpu.sync_copy(pv, xmsh.at[dstv], add=True)
        plsc.subcore_barrier()

        pltpu.sync_copy(xmsh.at[pl.ds(nbase, _NSLICE)], slicev)

        @pl.loop(0, _NSLICE // _LANES)
        def _(j):
            sl = pl.ds(j * _LANES, _LANES)
            slicev[sl] = slicev[sl] * _ALPHA

        pltpu.sync_copy(slicev, xout_hbm.at[pl.ds(nbase, _NSLICE)])
        pltpu.sync_copy(binitv, xmsh.at[pl.ds(nbase, _NSLICE)])
        plsc.subcore_barrier()
        pltpu.sync_copy(xout_hbm, x_vmem)


_cp = pltpu.CompilerParams()
if "needs_layout_passes" in pltpu.CompilerParams.__dataclass_fields__:
    _cp = dataclasses.replace(_cp, needs_layout_passes=False)

_pr_call = functools.partial(
    pl.kernel,
    compiler_params=_cp,
    out_type=(jax.ShapeDtypeStruct((_NPAD,), jnp.float32),
              jax.ShapeDtypeStruct((_EPAD,), jnp.float32)),
    mesh=plsc.VectorSubcoreMesh(core_axis_name="c", subcore_axis_name="s",
                                num_cores=1),
    scratch_types=[
        pltpu.VMEM((_NPAD,), jnp.float32),
        pltpu.VMEM((_BLK,), jnp.int32),
        pltpu.VMEM((_BLK,), jnp.int32),
        pltpu.VMEM((_BLK,), jnp.float32),
        pltpu.VMEM((_BLK,), jnp.float32),
        pltpu.VMEM((_NSRC,), jnp.int32),
        pltpu.VMEM((_NSRC,), jnp.float32),
        pltpu.VMEM((_NSLICE,), jnp.float32),
        pltpu.VMEM((_NSLICE,), jnp.float32),
        pltpu.VMEM_SHARED((_NPAD,), jnp.float32),
    ],
)(_body)


def kernel(edge_index, edge_weight, source_nodes):
    src = edge_index[0]
    dst = edge_index[1]
    pad = _EPAD - _E
    fill = jnp.arange(pad, dtype=jnp.int32) % _N
    src1 = jnp.concatenate([src, fill])
    dst1 = jnp.concatenate([dst, fill])
    ew1 = jnp.concatenate([edge_weight, jnp.zeros((pad,), jnp.float32)])
    xpad, _ = _pr_call(src1, dst1, ew1, source_nodes)
    return xpad[:_N]


# speedup vs baseline: 197.6674x; 2.5233x over previous
import dataclasses
import functools

import jax
import jax.numpy as jnp
from jax import lax
from jax.experimental import pallas as pl
from jax.experimental.pallas import tpu as pltpu
from jax.experimental.pallas import tpu_sc as plsc

_N = 100000
_E = 1600000
_ALPHA = 0.85
_ITERS = 100
_NSRC = 256

_LANES = 16
_TILES = 16
_BLK = 1024
_NBLK = 100
_EPT = _BLK * _NBLK
_EPAD = _EPT * _TILES
_NPAD = 100096
_NSLICE = _NPAD // _TILES
_BINIT = (1.0 - _ALPHA) / (_ALPHA * _NSRC)


def _f32x16(v):
    return jnp.full((_LANES,), v, dtype=jnp.float32)


def _body(src_hbm, dst_hbm, ew_hbm, srcn_hbm, xout_hbm, w_hbm,
          x_vmem, srcv, dstv, wv, pv, srcv1, dstv1, wv1, pv1,
          srcnv, biasv, binitv, slicev, xmsh, sem_in0, sem_in1,
          sem_sc0, sem_sc1):
    sid = lax.axis_index("s")
    ebase = sid * _EPT
    nbase = sid * _NSLICE

    bufs = ((srcv, dstv, wv, pv, sem_in0, sem_sc0),
            (srcv1, dstv1, wv1, pv1, sem_in1, sem_sc1))

    def issue_in(b, s):
        sb, db, wb, _, sem, _ = bufs[s]
        e0 = ebase + b * _BLK
        pltpu.async_copy(src_hbm.at[pl.ds(e0, _BLK)], sb, sem)
        pltpu.async_copy(dst_hbm.at[pl.ds(e0, _BLK)], db, sem)
        pltpu.async_copy(w_hbm.at[pl.ds(e0, _BLK)], wb, sem)

    def wait_in(s):
        sb, db, wb, _, sem, _ = bufs[s]
        pltpu.make_async_copy(src_hbm.at[pl.ds(0, _BLK)], sb, sem).wait()
        pltpu.make_async_copy(dst_hbm.at[pl.ds(0, _BLK)], db, sem).wait()
        pltpu.make_async_copy(w_hbm.at[pl.ds(0, _BLK)], wb, sem).wait()

    def compute(s):
        sb, _, wb, pb, _, _ = bufs[s]
        for i in range(_BLK // _LANES):
            sl = pl.ds(i * _LANES, _LANES)
            xg = plsc.load_gather(x_vmem, [sb[sl]])
            pb[sl] = xg * wb[sl]

    def issue_sc(s):
        _, db, _, pb, _, sem = bufs[s]
        pltpu.async_copy(pb, xmsh.at[db], sem, add=True)

    def wait_sc(s):
        _, db, _, pb, _, sem = bufs[s]
        pltpu.make_async_copy(pb, xmsh.at[db], sem).wait()

    @pl.loop(0, _NSLICE // _LANES)
    def _(j):
        slicev[pl.ds(j * _LANES, _LANES)] = _f32x16(0.0)

    pltpu.sync_copy(slicev, xmsh.at[pl.ds(nbase, _NSLICE)])
    plsc.subcore_barrier()

    @pl.loop(0, _NBLK)
    def _(blk):
        e0 = ebase + blk * _BLK
        pltpu.sync_copy(src_hbm.at[pl.ds(e0, _BLK)], srcv)
        pltpu.sync_copy(ew_hbm.at[pl.ds(e0, _BLK)], wv)
        pltpu.sync_copy(wv, xmsh.at[srcv], add=True)
    plsc.subcore_barrier()

    pltpu.sync_copy(xmsh, x_vmem)
    plsc.subcore_barrier()

    pltpu.sync_copy(slicev, xmsh.at[pl.ds(nbase, _NSLICE)])
    plsc.subcore_barrier()

    @pl.when(sid == 0)
    def _():
        pltpu.sync_copy(srcn_hbm, srcnv)

        @pl.loop(0, _NSRC // _LANES)
        def _(k):
            biasv[pl.ds(k * _LANES, _LANES)] = _f32x16(_BINIT)

        pltpu.sync_copy(biasv, xmsh.at[srcnv], add=True)
    plsc.subcore_barrier()

    pltpu.sync_copy(xmsh.at[pl.ds(nbase, _NSLICE)], binitv)

    @pl.loop(0, _NBLK)
    def _(blk):
        e0 = ebase + blk * _BLK
        pltpu.sync_copy(src_hbm.at[pl.ds(e0, _BLK)], srcv)
        pltpu.sync_copy(ew_hbm.at[pl.ds(e0, _BLK)], wv)
        for i in range(_BLK // _LANES):
            sl = pl.ds(i * _LANES, _LANES)
            rs = plsc.load_gather(x_vmem, [srcv[sl]])
            nz = rs != _f32x16(0.0)
            safe = jnp.where(nz, rs, _f32x16(1.0))
            pv[sl] = jnp.where(nz, wv[sl] / safe, _f32x16(0.0))
        pltpu.sync_copy(pv, w_hbm.at[pl.ds(e0, _BLK)])

    @pl.loop(0, _NPAD // _LANES)
    def _(j):
        x_vmem[pl.ds(j * _LANES, _LANES)] = _f32x16(1.0 / _N)
    plsc.subcore_barrier()

    @pl.loop(0, _ITERS)
    def _(it):
        issue_in(0, 0)

        @pl.loop(0, _NBLK // 2)
        def _(h):
            b0 = 2 * h
            issue_in(b0 + 1, 1)
            wait_in(0)

            @pl.when(h > 0)
            def _():
                wait_sc(0)

            compute(0)
            issue_sc(0)

            @pl.when(h + 1 < _NBLK // 2)
            def _():
                issue_in(b0 + 2, 0)

            wait_in(1)

            @pl.when(h > 0)
            def _():
                wait_sc(1)

            compute(1)
            issue_sc(1)

        wait_sc(0)
        wait_sc(1)
        plsc.subcore_barrier()

        pltpu.sync_copy(xmsh.at[pl.ds(nbase, _NSLICE)], slicev)

        @pl.loop(0, _NSLICE // _LANES)
        def _(j):
            sl = pl.ds(j * _LANES, _LANES)
            slicev[sl] = slicev[sl] * _ALPHA

        pltpu.sync_copy(slicev, xout_hbm.at[pl.ds(nbase, _NSLICE)])
        pltpu.sync_copy(binitv, xmsh.at[pl.ds(nbase, _NSLICE)])
        plsc.subcore_barrier()
        pltpu.sync_copy(xout_hbm, x_vmem)


_cp = pltpu.CompilerParams()
if "needs_layout_passes" in pltpu.CompilerParams.__dataclass_fields__:
    _cp = dataclasses.replace(_cp, needs_layout_passes=False)

_pr_call = functools.partial(
    pl.kernel,
    compiler_params=_cp,
    out_type=(jax.ShapeDtypeStruct((_NPAD,), jnp.float32),
              jax.ShapeDtypeStruct((_EPAD,), jnp.float32)),
    mesh=plsc.VectorSubcoreMesh(core_axis_name="c", subcore_axis_name="s",
                                num_cores=1),
    scratch_types=[
        pltpu.VMEM((_NPAD,), jnp.float32),
        pltpu.VMEM((_BLK,), jnp.int32),
        pltpu.VMEM((_BLK,), jnp.int32),
        pltpu.VMEM((_BLK,), jnp.float32),
        pltpu.VMEM((_BLK,), jnp.float32),
        pltpu.VMEM((_BLK,), jnp.int32),
        pltpu.VMEM((_BLK,), jnp.int32),
        pltpu.VMEM((_BLK,), jnp.float32),
        pltpu.VMEM((_BLK,), jnp.float32),
        pltpu.VMEM((_NSRC,), jnp.int32),
        pltpu.VMEM((_NSRC,), jnp.float32),
        pltpu.VMEM((_NSLICE,), jnp.float32),
        pltpu.VMEM((_NSLICE,), jnp.float32),
        pltpu.VMEM_SHARED((_NPAD,), jnp.float32),
        pltpu.SemaphoreType.DMA,
        pltpu.SemaphoreType.DMA,
        pltpu.SemaphoreType.DMA,
        pltpu.SemaphoreType.DMA,
    ],
)(_body)


def kernel(edge_index, edge_weight, source_nodes):
    src = edge_index[0]
    dst = edge_index[1]
    pad = _EPAD - _E
    fill = jnp.arange(pad, dtype=jnp.int32) % _N
    src1 = jnp.concatenate([src, fill])
    dst1 = jnp.concatenate([dst, fill])
    ew1 = jnp.concatenate([edge_weight, jnp.zeros((pad,), jnp.float32)])
    xpad, _ = _pr_call(src1, dst1, ew1, source_nodes)
    return xpad[:_N]
